# Initial kernel scaffold; baseline (speedup 1.0000x reference)
import jax
import jax.numpy as jnp
from jax.experimental import pallas as pl


def kernel(x, batch, phi_w1, phi_b1, phi_w2, phi_b2, rho_w1, rho_b1, rho_w2, rho_b2):
    raise NotImplementedError("write your pallas kernel here")


# speedup vs baseline: 2.1120x; 2.1120x over previous
import functools

import jax
import jax.numpy as jnp
from jax import lax
from jax.experimental import pallas as pl
from jax.experimental.pallas import tpu as pltpu
from jax.experimental.pallas import tpu_sc as plsc

N = 320000
D = 128
NUM_SEG = 10000
SEG_PAD = 10240

NW = 32
BR = 128
NB = N // BR
NB_PAD = 2560
BPW = NB_PAD // NW
SEG_PER_TILE = SEG_PAD // 16

_sc_mesh = plsc.VectorSubcoreMesh(core_axis_name="c", subcore_axis_name="s")


def _segsum_body(x_hbm, b2d_hbm, zeros_hbm, out_hbm, idx_v, xb_v, xsum_sh):
    c = lax.axis_index("c")
    s = lax.axis_index("s")
    wid = c * 16 + s
    seg0 = pl.multiple_of(s * SEG_PER_TILE, 8)

    pltpu.sync_copy(zeros_hbm, xsum_sh.at[pl.ds(seg0, SEG_PER_TILE)])
    pltpu.sync_copy(b2d_hbm.at[pl.ds(pl.multiple_of(wid * BPW, 8), BPW)], idx_v)
    plsc.subcore_barrier()

    def block(j, carry):
        b = wid * BPW + j

        @pl.when(b < NB)
        def _():
            row0 = pl.multiple_of(b * BR, 8)
            pltpu.sync_copy(x_hbm.at[pl.ds(row0, BR)], xb_v)
            pltpu.sync_copy(xb_v, xsum_sh.at[idx_v.at[j]], add=True)

        return carry

    lax.fori_loop(0, BPW, block, 0)
    plsc.subcore_barrier()

    pltpu.sync_copy(
        xsum_sh.at[pl.ds(seg0, SEG_PER_TILE)],
        out_hbm.at[c, pl.ds(seg0, SEG_PER_TILE)],
    )


_segsum_call = functools.partial(
    pl.kernel,
    out_type=jax.ShapeDtypeStruct((2, SEG_PAD, D), jnp.float32),
    mesh=_sc_mesh,
    scratch_types=[
        pltpu.VMEM((BPW, BR), jnp.int32),
        pltpu.VMEM((BR, D), jnp.float32),
        pltpu.VMEM_SHARED((SEG_PAD, D), jnp.float32),
    ],
)(_segsum_body)


def _gather_add_body(phi_hbm, rho_hbm, b2d_hbm, out_hbm, idx_v, xb_v, gb_v, sem):
    c = lax.axis_index("c")
    s = lax.axis_index("s")
    wid = c * 16 + s

    pltpu.sync_copy(b2d_hbm.at[pl.ds(pl.multiple_of(wid * BPW, 8), BPW)], idx_v)

    def block(j, carry):
        b = wid * BPW + j

        @pl.when(b < NB)
        def _():
            row0 = pl.multiple_of(b * BR, 8)
            pltpu.async_copy(rho_hbm.at[idx_v.at[j]], gb_v, sem).wait()
            pltpu.sync_copy(phi_hbm.at[pl.ds(row0, BR)], xb_v)

            def addrow(r, carry2):
                for cc in range(D // 16):
                    sl = pl.ds(cc * 16, 16)
                    xb_v[r, sl] = xb_v[r, sl] + gb_v[r, sl]
                return carry2

            lax.fori_loop(0, BR, addrow, 0)
            pltpu.sync_copy(xb_v, out_hbm.at[pl.ds(row0, BR)])

        return carry

    lax.fori_loop(0, BPW, block, 0)


_gather_add_call = functools.partial(
    pl.kernel,
    out_type=jax.ShapeDtypeStruct((N, D), jnp.float32),
    mesh=_sc_mesh,
    scratch_types=[
        pltpu.VMEM((BPW, BR), jnp.int32),
        pltpu.VMEM((BR, D), jnp.float32),
        pltpu.VMEM((BR, D), jnp.float32),
        pltpu.SemaphoreType.DMA,
    ],
)(_gather_add_body)


PHI_ROWS = 2000


def _phi_body(x_ref, w1_ref, b1_ref, w2_ref, b2_ref, o_ref):
    h = jnp.maximum(
        jnp.dot(x_ref[...], w1_ref[...], preferred_element_type=jnp.float32)
        + b1_ref[...],
        0.0,
    )
    o_ref[...] = (
        jnp.dot(h, w2_ref[...], preferred_element_type=jnp.float32) + b2_ref[...]
    )


def _phi_call(x, w1, b1, w2, b2):
    grid = (N // PHI_ROWS,)
    return pl.pallas_call(
        _phi_body,
        grid=grid,
        in_specs=[
            pl.BlockSpec((PHI_ROWS, D), lambda i: (i, 0)),
            pl.BlockSpec((D, D), lambda i: (0, 0)),
            pl.BlockSpec((1, D), lambda i: (0, 0)),
            pl.BlockSpec((D, D), lambda i: (0, 0)),
            pl.BlockSpec((1, D), lambda i: (0, 0)),
        ],
        out_specs=pl.BlockSpec((PHI_ROWS, D), lambda i: (i, 0)),
        out_shape=jax.ShapeDtypeStruct((N, D), jnp.float32),
    )(x, w1, b1, w2, b2)


RHO_ROWS = 2000


def _rho_body(p_ref, w1_ref, b1_ref, w2_ref, b2_ref, o_ref):
    xs = p_ref[0] + p_ref[1]
    h = jnp.maximum(
        jnp.dot(xs, w1_ref[...], preferred_element_type=jnp.float32) + b1_ref[...],
        0.0,
    )
    o_ref[...] = (
        jnp.dot(h, w2_ref[...], preferred_element_type=jnp.float32) + b2_ref[...]
    )


def _rho_call(partials, w1, b1, w2, b2):
    grid = (NUM_SEG // RHO_ROWS,)
    return pl.pallas_call(
        _rho_body,
        grid=grid,
        in_specs=[
            pl.BlockSpec((2, RHO_ROWS, D), lambda i: (0, i, 0)),
            pl.BlockSpec((D, D), lambda i: (0, 0)),
            pl.BlockSpec((1, D), lambda i: (0, 0)),
            pl.BlockSpec((D, D), lambda i: (0, 0)),
            pl.BlockSpec((1, D), lambda i: (0, 0)),
        ],
        out_specs=pl.BlockSpec((RHO_ROWS, D), lambda i: (i, 0)),
        out_shape=jax.ShapeDtypeStruct((NUM_SEG, D), jnp.float32),
    )(partials, w1, b1, w2, b2)


def kernel(x, batch, phi_w1, phi_b1, phi_w2, phi_b2, rho_w1, rho_b1, rho_w2, rho_b2):
    bi = batch.astype(jnp.int32)
    b2d = jnp.concatenate(
        [bi, jnp.zeros((NB_PAD * BR - N,), jnp.int32)]
    ).reshape(NB_PAD, BR)
    zeros_tile = jnp.zeros((SEG_PER_TILE, D), jnp.float32)

    x_phi = _phi_call(x, phi_w1, phi_b1.reshape(1, D), phi_w2, phi_b2.reshape(1, D))
    partials = _segsum_call(x, b2d, zeros_tile)
    rho_out = _rho_call(
        partials, rho_w1, rho_b1.reshape(1, D), rho_w2, rho_b2.reshape(1, D)
    )
    return _gather_add_call(x_phi, rho_out, b2d)
